# baseline (speedup 1.0000x reference)
import functools

import jax
import jax.numpy as jnp
from jax.experimental import pallas as pl
from jax.experimental.pallas import tpu as pltpu

_ONES_W = 128


def _p1_kernel(a_ref, x_ref, m1_ref, invn_ref, s_scr, *, nsteps, d):
    i = pl.program_id(0)
    y = x_ref[...]
    y2 = (y * y).astype(jnp.bfloat16)
    ones = jnp.ones((y.shape[0], _ONES_W), jnp.bfloat16)
    y_aug = jnp.concatenate([y2, ones], axis=1)
    a = a_ref[...]
    part = jax.lax.dot_general(
        a, y_aug, (((0,), (0,)), ((), ())), preferred_element_type=jnp.float32
    )

    @pl.when(i == 0)
    def _init():
        s_scr[...] = part

    @pl.when(i > 0)
    def _acc():
        s_scr[...] += part

    @pl.when(i == nsteps - 1)
    def _finish():
        s = s_scr[:, :d]
        n = s_scr[:, d : d + 1]
        invn = 1.0 / jnp.maximum(n, 1.0)
        m1_ref[...] = (s * invn).astype(jnp.bfloat16)
        invn_ref[...] = invn


def _p2_kernel(a_ref, m1_ref, w1_ref, invn_ref, m2_ref, s_scr, *, nsteps):
    i = pl.program_id(0)
    a = a_ref[...]
    z = jnp.dot(a, m1_ref[...], preferred_element_type=jnp.float32)
    e = jnp.sum(a.astype(jnp.float32), axis=1, keepdims=True)
    u = jnp.sqrt(z * (1.0 / jnp.maximum(e, 1.0)))
    h = jnp.maximum(
        jnp.dot(u.astype(jnp.bfloat16), w1_ref[...], preferred_element_type=jnp.float32),
        0.0,
    )
    h2 = (h * h).astype(jnp.bfloat16)
    part = jax.lax.dot_general(
        a, h2, (((0,), (0,)), ((), ())), preferred_element_type=jnp.float32
    )

    @pl.when(i == 0)
    def _init():
        s_scr[...] = part

    @pl.when(i > 0)
    def _acc():
        s_scr[...] += part

    @pl.when(i == nsteps - 1)
    def _finish():
        m2_ref[...] = (s_scr[...] * invn_ref[...]).astype(jnp.bfloat16)


def _p3_kernel(a_ref, m2_ref, w2_ref, out_ref):
    a = a_ref[...]
    z = jnp.dot(a, m2_ref[...], preferred_element_type=jnp.float32)
    e = jnp.sum(a.astype(jnp.float32), axis=1, keepdims=True)
    u = jnp.sqrt(z * (1.0 / jnp.maximum(e, 1.0)))
    out_ref[...] = jnp.maximum(
        jnp.dot(u.astype(jnp.bfloat16), w2_ref[...], preferred_element_type=jnp.float32),
        0.0,
    )


def kernel(x_0, incidence, W1, W2):
    n, d = x_0.shape
    e = incidence.shape[1]
    bn = 2000
    nsteps = n // bn

    a16 = incidence.astype(jnp.bfloat16)
    w1_16 = W1.astype(jnp.bfloat16)
    w2_16 = W2.astype(jnp.bfloat16)

    m1, invn = pl.pallas_call(
        functools.partial(_p1_kernel, nsteps=nsteps, d=d),
        grid=(nsteps,),
        in_specs=[
            pl.BlockSpec((bn, e), lambda i: (i, 0)),
            pl.BlockSpec((bn, d), lambda i: (i, 0)),
        ],
        out_specs=[
            pl.BlockSpec((e, d), lambda i: (0, 0)),
            pl.BlockSpec((e, 1), lambda i: (0, 0)),
        ],
        out_shape=[
            jax.ShapeDtypeStruct((e, d), jnp.bfloat16),
            jax.ShapeDtypeStruct((e, 1), jnp.float32),
        ],
        scratch_shapes=[pltpu.VMEM((e, d + _ONES_W), jnp.float32)],
    )(a16, x_0)

    m2 = pl.pallas_call(
        functools.partial(_p2_kernel, nsteps=nsteps),
        grid=(nsteps,),
        in_specs=[
            pl.BlockSpec((bn, e), lambda i: (i, 0)),
            pl.BlockSpec((e, d), lambda i: (0, 0)),
            pl.BlockSpec((d, d), lambda i: (0, 0)),
            pl.BlockSpec((e, 1), lambda i: (0, 0)),
        ],
        out_specs=pl.BlockSpec((e, d), lambda i: (0, 0)),
        out_shape=jax.ShapeDtypeStruct((e, d), jnp.bfloat16),
        scratch_shapes=[pltpu.VMEM((e, d), jnp.float32)],
    )(a16, m1, w1_16, invn)

    out = pl.pallas_call(
        _p3_kernel,
        grid=(nsteps,),
        in_specs=[
            pl.BlockSpec((bn, e), lambda i: (i, 0)),
            pl.BlockSpec((e, d), lambda i: (0, 0)),
            pl.BlockSpec((d, d), lambda i: (0, 0)),
        ],
        out_specs=pl.BlockSpec((bn, d), lambda i: (i, 0)),
        out_shape=jax.ShapeDtypeStruct((n, d), jnp.float32),
    )(a16, m2, w2_16)

    return out
